# baseline (speedup 1.0000x reference)
import jax
import jax.numpy as jnp
from jax import lax
from jax.experimental import pallas as pl
from jax.experimental.pallas import tpu as pltpu
from jax.experimental.pallas import tpu_sc as plsc

NC = 2
NS = 16
LANES = 16
NW = NC * NS

D = 32
SLEN = 128
SPW = 8


def _mf_body(user_ref, item_ref, ut_ref, it_ref, out_ref,
             uidx_v, iidx_v, uwidx, iwidx, ubuf, vbuf, outv,
             sem_i, sem_u, sem_v):
    n_rows_tab = ut_ref.shape[0] // D
    b_per_w = uidx_v.shape[0]
    n_chunks = b_per_w // LANES
    n_streams = (b_per_w * D) // SLEN
    n_groups = b_per_w // LANES
    qpg = b_per_w // SLEN

    wid = lax.axis_index("s") * NC + lax.axis_index("c")
    base = wid * b_per_w

    cu = pltpu.async_copy(user_ref.at[pl.ds(base, b_per_w)], uidx_v, sem_i)
    ci = pltpu.async_copy(item_ref.at[pl.ds(base, b_per_w)], iidx_v, sem_i)
    cu.wait()
    ci.wait()

    def idx_body(j, _):
        qoff = j // (SLEN // LANES)
        coff = (j % (SLEN // LANES)) * LANES
        uvec = uidx_v[pl.ds(j * LANES, LANES)]
        ivec = iidx_v[pl.ds(j * LANES, LANES)]
        for c in range(D):
            uwidx[c * qpg + qoff, pl.ds(coff, LANES)] = uvec + c * n_rows_tab
            iwidx[c * qpg + qoff, pl.ds(coff, LANES)] = ivec + c * n_rows_tab
        return 0

    lax.fori_loop(0, n_chunks, idx_body, 0)

    def issue(w):
        for k in range(SPW):
            q = w * SPW + k
            pltpu.async_copy(ut_ref.at[uwidx.at[q]], ubuf.at[q], sem_u)
            pltpu.async_copy(it_ref.at[iwidx.at[q]], vbuf.at[q], sem_v)

    def drain():
        for _ in range(SPW):
            pltpu.make_async_copy(ut_ref.at[pl.ds(0, SLEN)],
                                  ubuf.at[0], sem_u).wait()
            pltpu.make_async_copy(it_ref.at[pl.ds(0, SLEN)],
                                  vbuf.at[0], sem_v).wait()

    n_batches = n_streams // SPW
    issue(0)

    def stream_body(w, _):
        issue(w)
        drain()
        return 0

    lax.fori_loop(1, n_batches, stream_body, 0)
    drain()

    def grp_body(g, _):
        qoff = g // (SLEN // LANES)
        coff = (g % (SLEN // LANES)) * LANES
        acc = (ubuf[qoff, pl.ds(coff, LANES)] *
               vbuf[qoff, pl.ds(coff, LANES)])
        for c in range(1, D):
            acc = acc + (ubuf[c * qpg + qoff, pl.ds(coff, LANES)] *
                         vbuf[c * qpg + qoff, pl.ds(coff, LANES)])
        outv[pl.ds(g * LANES, LANES)] = acc
        return 0

    lax.fori_loop(0, n_groups, grp_body, 0)

    pltpu.sync_copy(outv, out_ref.at[pl.ds(base, b_per_w)])


def _build(batch, n_rows):
    b_per_w = batch // NW
    n_streams = (b_per_w * D) // SLEN
    mesh = plsc.VectorSubcoreMesh(core_axis_name="c", subcore_axis_name="s")
    return pl.kernel(
        _mf_body,
        out_type=jax.ShapeDtypeStruct((batch,), jnp.float32),
        mesh=mesh,
        compiler_params=pltpu.CompilerParams(
            needs_layout_passes=False, use_tc_tiling_on_sc=False),
        scratch_types=[
            pltpu.VMEM((b_per_w,), jnp.int32),
            pltpu.VMEM((b_per_w,), jnp.int32),
            pltpu.VMEM((n_streams, SLEN), jnp.int32),
            pltpu.VMEM((n_streams, SLEN), jnp.int32),
            pltpu.VMEM((n_streams, SLEN), jnp.float32),
            pltpu.VMEM((n_streams, SLEN), jnp.float32),
            pltpu.VMEM((b_per_w,), jnp.float32),
            pltpu.SemaphoreType.DMA,
            pltpu.SemaphoreType.DMA,
            pltpu.SemaphoreType.DMA,
        ],
    )


@jax.jit
def kernel(user, item, user_table, item_table):
    batch = user.shape[0]
    n_rows = user_table.shape[0]
    ut_flat = user_table.T.reshape(-1)
    it_flat = item_table.T.reshape(-1)
    return _build(batch, n_rows)(user, item, ut_flat, it_flat)
